# Initial kernel scaffold; baseline (speedup 1.0000x reference)
import jax
import jax.numpy as jnp
from jax.experimental import pallas as pl


def kernel(X):
    raise NotImplementedError("write your pallas kernel here")


# speedup vs baseline: 27.9918x; 27.9918x over previous
import jax
import jax.numpy as jnp
from jax.experimental import pallas as pl


def _rr_body(x_ref, pi_ref):
    n, s, l = x_ref.shape
    m = s * l
    sub = jax.lax.broadcasted_iota(jnp.int32, (s, l), 0)
    lane = jax.lax.broadcasted_iota(jnp.int32, (s, l), 1)
    j_idx = sub * l + lane
    neg_inf = jnp.float32(-jnp.inf)

    def step(t, owner):
        row = jax.lax.rem(t, n)
        a = x_ref[pl.ds(row, 1), :, :].reshape(s, l)
        masked = jnp.where(owner >= 0, neg_inf, a)
        mx = jnp.max(masked)
        cand = jnp.where(masked == mx, j_idx, jnp.int32(m))
        idx = jnp.min(cand)
        return jnp.where(j_idx == idx, row, owner)

    owner0 = jnp.full((s, l), -1, jnp.int32)
    owner = jax.lax.fori_loop(0, m, step, owner0)
    rows = jax.lax.broadcasted_iota(jnp.int32, (n, s, l), 0)
    pi_ref[...] = (owner[None] == rows).astype(jnp.float32)


def _round_robin_2d(x):
    n, m = x.shape
    s = 8
    x3 = x.reshape(n, s, m // s)
    pi3 = pl.pallas_call(
        _rr_body,
        out_shape=jax.ShapeDtypeStruct((n, s, m // s), jnp.float32),
    )(x3)
    return pi3.reshape(n, m)


def kernel(X):
    if X.ndim == 2:
        return _round_robin_2d(X)
    return jnp.stack([_round_robin_2d(X[i]) for i in range(X.shape[0])])


# speedup vs baseline: 29.9449x; 1.0698x over previous
import functools

import jax
import jax.numpy as jnp
from jax import lax
from jax.experimental import pallas as pl
from jax.experimental.pallas import tpu as pltpu
from jax.experimental.pallas import tpu_sc as plsc

_NEG_INF = float("-inf")


def _round_robin_2d(x):
    n, m = x.shape
    num_tiles = 16
    apt = n // num_tiles
    ngroups = m // 16
    nsuper = ngroups // 16
    rounds = m // n

    mesh = plsc.VectorSubcoreMesh(
        core_axis_name="c", subcore_axis_name="s", num_cores=1
    )

    def body(x_hbm, out_hbm, x_loc, pi_loc, taken, g1val, g1idx, g2val,
             cand_loc, rbuf, ubuf, cand_slab, upto_slab):
        wid = lax.axis_index("s")
        base = wid * apt
        lane16 = lax.iota(jnp.int32, 16)
        lane0 = lane16 == 0

        def st1(ref, idxs, val):
            plsc.store_scatter(
                ref,
                [jnp.full((16,), i, jnp.int32) for i in idxs],
                jnp.full((16,), val, ref.dtype),
                mask=lane0,
            )

        def ld1(ref, idxs):
            v = plsc.load_gather(
                ref,
                [jnp.full((16,), i, jnp.int32) for i in idxs],
                mask=lane0,
            )
            return v[0]

        def ffs_scalar(mask_vec):
            return jnp.max(plsc.all_reduce_ffs(mask_vec))

        def recompute_group(b, g):
            lo = g * 16
            vals = x_loc[b, pl.ds(lo, 16)]
            tk = taken[pl.ds(lo, 16)]
            masked = jnp.where(tk == 0, vals, _NEG_INF)
            mx = jnp.max(masked)
            st1(g1val, (b, g), mx)
            st1(g1idx, (b, g), lo + ffs_scalar(masked == mx))
            s = lax.div(g, 16)
            sv = g1val[b, pl.ds(s * 16, 16)]
            st1(g2val, (b, s), jnp.max(sv))

        pltpu.sync_copy(x_hbm.at[pl.ds(base, apt)], x_loc)
        zf = jnp.zeros((16,), jnp.float32)
        zi = jnp.zeros((16,), jnp.int32)

        def zero_taken(i, _):
            taken[pl.ds(i * 16, 16)] = zi
            return 0

        lax.fori_loop(0, m // 16, zero_taken, 0)

        def zero_pi(i, _):
            b = lax.div(i, m // 16)
            o = lax.rem(i, m // 16)
            pi_loc[b, pl.ds(o * 16, 16)] = zf
            return 0

        lax.fori_loop(0, apt * (m // 16), zero_pi, 0)

        def build_g1(i, _):
            recompute_group(lax.div(i, ngroups), lax.rem(i, ngroups))
            return 0

        lax.fori_loop(0, apt * ngroups, build_g1, 0)

        def round_body(r, _):
            def sr_cond(done):
                return done < n

            def sr_body(done):
                for b in range(apt):
                    @pl.when(base + b >= done)
                    def _():
                        g2 = g2val[b, :]
                        s = ffs_scalar(g2 == jnp.max(g2))
                        grp = g1val[b, pl.ds(s * 16, 16)]
                        g = s * 16 + ffs_scalar(grp == jnp.max(grp))
                        st1(cand_loc, (b,), ld1(g1idx, (b, g)))

                pltpu.sync_copy(cand_loc, cand_slab.at[wid])
                plsc.subcore_barrier()
                pltpu.sync_copy(cand_slab, rbuf)

                @pl.when(wid == 0)
                def _():
                    def rcond(u):
                        uc = jnp.minimum(u, n - 1)
                        item = ld1(rbuf, (lax.div(uc, apt), lax.rem(uc, apt)))
                        return jnp.logical_and(u < n, ld1(taken, (item,)) == 0)

                    def rbody(u):
                        item = ld1(rbuf, (lax.div(u, apt), lax.rem(u, apt)))
                        st1(taken, (item,), 1)
                        return u + 1

                    upto = lax.while_loop(rcond, rbody, done)
                    ubuf[:] = jnp.full((16,), upto, jnp.int32)
                    pltpu.sync_copy(ubuf, upto_slab)

                plsc.subcore_barrier()
                pltpu.sync_copy(upto_slab, ubuf)
                upto = jnp.max(ubuf[:])

                def apply_pick(a, _):
                    item = ld1(rbuf, (lax.div(a, apt), lax.rem(a, apt)))
                    st1(taken, (item,), 1)

                    @pl.when(jnp.logical_and(a >= base, a < base + apt))
                    def _():
                        st1(pi_loc, (a - base, item), 1.0)

                    gj = lax.div(item, 16)
                    for b in range(apt):
                        @pl.when(ld1(g1idx, (b, gj)) == item)
                        def _():
                            recompute_group(b, gj)

                    return 0

                lax.fori_loop(done, upto, apply_pick, 0)
                plsc.subcore_barrier()
                return upto

            lax.while_loop(sr_cond, sr_body, 0)
            return 0

        lax.fori_loop(0, rounds, round_body, 0)
        pltpu.sync_copy(pi_loc, out_hbm.at[pl.ds(base, apt)])

    run = pl.kernel(
        body,
        out_type=jax.ShapeDtypeStruct((n, m), jnp.float32),
        mesh=mesh,
        compiler_params=pltpu.CompilerParams(
            needs_layout_passes=False, use_tc_tiling_on_sc=False
        ),
        scratch_types=[
            pltpu.VMEM((apt, m), jnp.float32),
            pltpu.VMEM((apt, m), jnp.float32),
            pltpu.VMEM((m,), jnp.int32),
            pltpu.VMEM((apt, ngroups), jnp.float32),
            pltpu.VMEM((apt, ngroups), jnp.int32),
            pltpu.VMEM((apt, nsuper), jnp.float32),
            pltpu.VMEM((16,), jnp.int32),
            pltpu.VMEM((num_tiles, 16), jnp.int32),
            pltpu.VMEM((16,), jnp.int32),
            pltpu.VMEM_SHARED((num_tiles, 16), jnp.int32),
            pltpu.VMEM_SHARED((16,), jnp.int32),
        ],
    )
    return run(x)


def kernel(X):
    if X.ndim == 2:
        return _round_robin_2d(X)
    return jnp.stack([_round_robin_2d(X[i]) for i in range(X.shape[0])])


# speedup vs baseline: 90.8461x; 3.0338x over previous
import functools

import jax
import jax.numpy as jnp
from jax import lax
from jax.experimental import pallas as pl
from jax.experimental.pallas import tpu as pltpu
from jax.experimental.pallas import tpu_sc as plsc

_NEG_INF = float("-inf")


def _round_robin_2d(x):
    n, m = x.shape
    num_tiles = 16
    apt = n // num_tiles
    ngroups = m // 16
    nsuper = ngroups // 16
    rounds = m // n

    mesh = plsc.VectorSubcoreMesh(
        core_axis_name="c", subcore_axis_name="s", num_cores=1
    )

    def body(x_hbm, out_hbm, x_loc, pi_loc, taken, cnt, g1val, g1idx, g2val,
             cand_loc, rbuf, cand_slab):
        wid = lax.axis_index("s")
        base = wid * apt
        lane16 = lax.iota(jnp.int32, 16)
        lane0 = lane16 == 0

        def st1(ref, idxs, val):
            plsc.store_scatter(
                ref,
                [jnp.full((16,), i, jnp.int32) for i in idxs],
                jnp.full((16,), val, ref.dtype),
                mask=lane0,
            )

        def ld1(ref, idxs):
            v = plsc.load_gather(
                ref,
                [jnp.full((16,), i, jnp.int32) for i in idxs],
                mask=lane0,
            )
            return v[0]

        def ffs_scalar(mask_vec):
            return jnp.max(plsc.all_reduce_ffs(mask_vec))

        def recompute_group(b, g):
            lo = g * 16
            vals = x_loc[b, pl.ds(lo, 16)]
            tk = taken[pl.ds(lo, 16)]
            masked = jnp.where(tk == 0, vals, _NEG_INF)
            mx = jnp.max(masked)
            st1(g1val, (b, g), mx)
            st1(g1idx, (b, g), lo + ffs_scalar(masked == mx))
            s = lax.div(g, 16)
            sv = g1val[b, pl.ds(s * 16, 16)]
            st1(g2val, (b, s), jnp.max(sv))

        pltpu.sync_copy(x_hbm.at[pl.ds(base, apt)], x_loc)
        zf = jnp.zeros((16,), jnp.float32)
        zi = jnp.zeros((16,), jnp.int32)

        def zero_taken(i, _):
            taken[pl.ds(i * 16, 16)] = zi
            cnt[pl.ds(i * 16, 16)] = zi
            return 0

        lax.fori_loop(0, m // 16, zero_taken, 0)

        def zero_pi(i, _):
            b = lax.div(i, m // 16)
            o = lax.rem(i, m // 16)
            pi_loc[b, pl.ds(o * 16, 16)] = zf
            return 0

        lax.fori_loop(0, apt * (m // 16), zero_pi, 0)

        def build_g1(i, _):
            recompute_group(lax.div(i, ngroups), lax.rem(i, ngroups))
            return 0

        lax.fori_loop(0, apt * ngroups, build_g1, 0)

        ones16 = jnp.ones((16,), jnp.int32)

        def round_body(r, parity):
            def sr_cond(c):
                return c[0] < n

            def sr_body(c):
                done, par = c
                for b in range(apt):
                    @pl.when(base + b >= done)
                    def _():
                        g2 = g2val[b, :]
                        s = ffs_scalar(g2 == jnp.max(g2))
                        grp = g1val[b, pl.ds(s * 16, 16)]
                        g = s * 16 + ffs_scalar(grp == jnp.max(grp))
                        st1(cand_loc, (b,), ld1(g1idx, (b, g)))

                pltpu.sync_copy(cand_loc, cand_slab.at[par, wid])
                plsc.subcore_barrier()
                pltpu.sync_copy(cand_slab.at[par], rbuf)

                def ch_cond(cc):
                    return cc[1] != 0

                def ch_body(cc):
                    u, _ = cc
                    a_vec = u + lane16
                    valid = a_vec < n
                    items = plsc.load_gather(
                        rbuf,
                        [lax.shift_right_logical(a_vec, 2),
                         lax.bitwise_and(a_vec, 3)],
                        mask=valid,
                    )
                    tkn = plsc.load_gather(taken, [items], mask=valid)
                    plsc.addupdate_scatter(cnt, [items], ones16, mask=valid)
                    mult = plsc.load_gather(cnt, [items], mask=valid)
                    plsc.store_scatter(cnt, [items], zi, mask=valid)
                    bad = jnp.logical_and(
                        jnp.logical_and(
                            jnp.logical_or(tkn != 0, mult > 1), valid
                        ),
                        jnp.logical_or(lane16 > 0, u > done),
                    )
                    stop = jnp.min(jnp.where(bad, lane16, 16))
                    acc_mask = jnp.logical_and(lane16 < stop, valid)
                    plsc.store_scatter(taken, [items], ones16, mask=acc_mask)
                    for b in range(apt):
                        leaders = plsc.load_gather(
                            g1idx,
                            [jnp.full((16,), b, jnp.int32),
                             lax.shift_right_logical(items, 4)],
                            mask=acc_mask,
                        )
                        hit = jnp.logical_and(leaders == items, acc_mask)

                        def rep_cond(cur):
                            return jnp.max(cur) >= 0

                        def rep_body(cur):
                            it = jnp.max(cur)
                            recompute_group(b, lax.shift_right_logical(it, 4))
                            return jnp.where(cur == it, -1, cur)

                        lax.while_loop(
                            rep_cond, rep_body,
                            jnp.where(hit, items, -1),
                        )

                    nacc = jnp.minimum(stop, n - u)
                    cont = jnp.logical_and(stop == 16, u + nacc < n)
                    return u + nacc, jnp.where(cont, 1, 0).astype(jnp.int32)

                upto, _ = lax.while_loop(ch_cond, ch_body,
                                         (done, jnp.int32(1)))

                for b in range(apt):
                    @pl.when(jnp.logical_and(base + b >= done,
                                             base + b < upto))
                    def _():
                        st1(pi_loc, (b, ld1(cand_loc, (b,))), 1.0)

                return upto, 1 - par

            _, parity = lax.while_loop(sr_cond, sr_body,
                                       (jnp.int32(0), parity))
            return parity

        lax.fori_loop(0, rounds, round_body, jnp.int32(0))
        pltpu.sync_copy(pi_loc, out_hbm.at[pl.ds(base, apt)])

    run = pl.kernel(
        body,
        out_type=jax.ShapeDtypeStruct((n, m), jnp.float32),
        mesh=mesh,
        compiler_params=pltpu.CompilerParams(
            needs_layout_passes=False, use_tc_tiling_on_sc=False
        ),
        scratch_types=[
            pltpu.VMEM((apt, m), jnp.float32),
            pltpu.VMEM((apt, m), jnp.float32),
            pltpu.VMEM((m,), jnp.int32),
            pltpu.VMEM((m,), jnp.int32),
            pltpu.VMEM((apt, ngroups), jnp.float32),
            pltpu.VMEM((apt, ngroups), jnp.int32),
            pltpu.VMEM((apt, nsuper), jnp.float32),
            pltpu.VMEM((16,), jnp.int32),
            pltpu.VMEM((num_tiles, 16), jnp.int32),
            pltpu.VMEM_SHARED((2, num_tiles, 16), jnp.int32),
        ],
    )
    return run(x)


def kernel(X):
    if X.ndim == 2:
        return _round_robin_2d(X)
    return jnp.stack([_round_robin_2d(X[i]) for i in range(X.shape[0])])


# speedup vs baseline: 92.2695x; 1.0157x over previous
import functools

import jax
import jax.numpy as jnp
from jax import lax
from jax.experimental import pallas as pl
from jax.experimental.pallas import tpu as pltpu
from jax.experimental.pallas import tpu_sc as plsc

_NEG_INF = float("-inf")


def _round_robin_2d(x):
    n, m = x.shape
    num_tiles = 16
    apt = n // num_tiles
    ngroups = m // 16
    nsuper = ngroups // 16
    rounds = m // n

    mesh = plsc.VectorSubcoreMesh(
        core_axis_name="c", subcore_axis_name="s", num_cores=1
    )

    def body(x_hbm, out_hbm, x_loc, pi_loc, taken, cnt, g1val, g1idx, g2val,
             cand_loc, rbuf, dirty, cand_slab):
        wid = lax.axis_index("s")
        base = wid * apt
        lane16 = lax.iota(jnp.int32, 16)
        lane0 = lane16 == 0

        def st1(ref, idxs, val):
            plsc.store_scatter(
                ref,
                [jnp.full((16,), i, jnp.int32) for i in idxs],
                jnp.full((16,), val, ref.dtype),
                mask=lane0,
            )

        def ld1(ref, idxs):
            v = plsc.load_gather(
                ref,
                [jnp.full((16,), i, jnp.int32) for i in idxs],
                mask=lane0,
            )
            return v[0]

        def ffs_scalar(mask_vec):
            return jnp.max(plsc.all_reduce_ffs(mask_vec))

        def recompute_group(b, g):
            lo = g * 16
            vals = x_loc[b, pl.ds(lo, 16)]
            tk = taken[pl.ds(lo, 16)]
            masked = jnp.where(tk == 0, vals, _NEG_INF)
            mx = jnp.max(masked)
            st1(g1val, (b, g), mx)
            st1(g1idx, (b, g), lo + ffs_scalar(masked == mx))
            s = lax.div(g, 16)
            sv = g1val[b, pl.ds(s * 16, 16)]
            st1(g2val, (b, s), jnp.max(sv))
            st1(dirty, (b,), 1)

        pltpu.sync_copy(x_hbm.at[pl.ds(base, apt)], x_loc)
        zf = jnp.zeros((16,), jnp.float32)
        zi = jnp.zeros((16,), jnp.int32)

        def zero_taken(i, _):
            taken[pl.ds(i * 16, 16)] = zi
            cnt[pl.ds(i * 16, 16)] = zi
            return 0

        lax.fori_loop(0, m // 16, zero_taken, 0)

        def zero_pi(i, _):
            b = lax.div(i, m // 16)
            o = lax.rem(i, m // 16)
            pi_loc[b, pl.ds(o * 16, 16)] = zf
            return 0

        lax.fori_loop(0, apt * (m // 16), zero_pi, 0)

        def build_g1(i, _):
            recompute_group(lax.div(i, ngroups), lax.rem(i, ngroups))
            return 0

        lax.fori_loop(0, apt * ngroups, build_g1, 0)

        ones16 = jnp.ones((16,), jnp.int32)

        def main_loop(parity):
            def sr_cond(c):
                return c[0] < m

            def sr_body(c):
                done, par = c
                dv = dirty[:]
                for b in range(apt):
                    @pl.when(dv[b] != 0)
                    def _():
                        g2 = g2val[b, :]
                        s = ffs_scalar(g2 == jnp.max(g2))
                        grp = g1val[b, pl.ds(s * 16, 16)]
                        g = s * 16 + ffs_scalar(grp == jnp.max(grp))
                        st1(cand_loc, (b,), ld1(g1idx, (b, g)))

                dirty[:] = zi
                pltpu.sync_copy(cand_loc, cand_slab.at[par, wid])
                plsc.subcore_barrier()
                pltpu.sync_copy(cand_slab.at[par], rbuf)

                limit = jnp.minimum(done + n, m)

                def ch_cond(cc):
                    return cc[1] != 0

                def ch_body(cc):
                    u, _ = cc
                    p_vec = u + lane16
                    valid = p_vec < limit
                    a_vec = lax.bitwise_and(p_vec, n - 1)
                    items = plsc.load_gather(
                        rbuf,
                        [lax.shift_right_logical(a_vec, 2),
                         lax.bitwise_and(a_vec, 3)],
                        mask=valid,
                    )
                    tkn = plsc.load_gather(taken, [items], mask=valid)
                    plsc.addupdate_scatter(cnt, [items], ones16, mask=valid)
                    mult = plsc.load_gather(cnt, [items], mask=valid)
                    plsc.store_scatter(cnt, [items], zi, mask=valid)
                    bad = jnp.logical_and(
                        jnp.logical_and(
                            jnp.logical_or(tkn != 0, mult > 1), valid
                        ),
                        jnp.logical_or(lane16 > 0, u > done),
                    )
                    stop = jnp.min(jnp.where(bad, lane16, 16))
                    acc_mask = jnp.logical_and(lane16 < stop, valid)
                    plsc.store_scatter(taken, [items], ones16, mask=acc_mask)
                    for b in range(apt):
                        leaders = plsc.load_gather(
                            g1idx,
                            [jnp.full((16,), b, jnp.int32),
                             lax.shift_right_logical(items, 4)],
                            mask=acc_mask,
                        )
                        hit = jnp.logical_and(leaders == items, acc_mask)

                        def rep_cond(cur):
                            return jnp.max(cur) >= 0

                        def rep_body(cur):
                            it = jnp.max(cur)
                            recompute_group(b, lax.shift_right_logical(it, 4))
                            return jnp.where(cur == it, -1, cur)

                        lax.while_loop(
                            rep_cond, rep_body,
                            jnp.where(hit, items, -1),
                        )

                    nacc = jnp.minimum(stop, limit - u)
                    cont = jnp.logical_and(stop == 16, u + nacc < limit)
                    return u + nacc, jnp.where(cont, 1, 0).astype(jnp.int32)

                upto, _ = lax.while_loop(ch_cond, ch_body,
                                         (done, jnp.int32(1)))

                for b in range(apt):
                    p_b = done + lax.bitwise_and(base + b - done, n - 1)

                    @pl.when(p_b < upto)
                    def _():
                        st1(pi_loc, (b, ld1(cand_loc, (b,))), 1.0)

                return upto, 1 - par

            lax.while_loop(sr_cond, sr_body, (jnp.int32(0), parity))

        main_loop(jnp.int32(0))
        pltpu.sync_copy(pi_loc, out_hbm.at[pl.ds(base, apt)])

    run = pl.kernel(
        body,
        out_type=jax.ShapeDtypeStruct((n, m), jnp.float32),
        mesh=mesh,
        compiler_params=pltpu.CompilerParams(
            needs_layout_passes=False, use_tc_tiling_on_sc=False
        ),
        scratch_types=[
            pltpu.VMEM((apt, m), jnp.float32),
            pltpu.VMEM((apt, m), jnp.float32),
            pltpu.VMEM((m,), jnp.int32),
            pltpu.VMEM((m,), jnp.int32),
            pltpu.VMEM((apt, ngroups), jnp.float32),
            pltpu.VMEM((apt, ngroups), jnp.int32),
            pltpu.VMEM((apt, nsuper), jnp.float32),
            pltpu.VMEM((16,), jnp.int32),
            pltpu.VMEM((num_tiles, 16), jnp.int32),
            pltpu.VMEM((16,), jnp.int32),
            pltpu.VMEM_SHARED((2, num_tiles, 16), jnp.int32),
        ],
    )
    return run(x)


def kernel(X):
    if X.ndim == 2:
        return _round_robin_2d(X)
    return jnp.stack([_round_robin_2d(X[i]) for i in range(X.shape[0])])


# speedup vs baseline: 101.1128x; 1.0958x over previous
import functools

import jax
import jax.numpy as jnp
from jax import lax
from jax.experimental import pallas as pl
from jax.experimental.pallas import tpu as pltpu
from jax.experimental.pallas import tpu_sc as plsc

_NEG_INF = float("-inf")


def _round_robin_2d(x):
    n, m = x.shape
    num_tiles = 16
    apt = n // num_tiles
    ngroups = m // 16
    nsuper = ngroups // 16
    rounds = m // n

    mesh = plsc.VectorSubcoreMesh(
        core_axis_name="c", subcore_axis_name="s", num_cores=1
    )

    def body(x_hbm, out_hbm, x_loc, pi_loc, taken, cnt, g1val, g1idx, g2val,
             cand_loc, rbuf, dirty, cand_slab):
        wid = lax.axis_index("s")
        base = wid * apt
        lane16 = lax.iota(jnp.int32, 16)
        lane0 = lane16 == 0

        def st1(ref, idxs, val):
            plsc.store_scatter(
                ref,
                [jnp.full((16,), i, jnp.int32) for i in idxs],
                jnp.full((16,), val, ref.dtype),
                mask=lane0,
            )

        def ld1(ref, idxs):
            v = plsc.load_gather(
                ref,
                [jnp.full((16,), i, jnp.int32) for i in idxs],
                mask=lane0,
            )
            return v[0]

        def ffs_scalar(mask_vec):
            return jnp.max(plsc.all_reduce_ffs(mask_vec))

        def recompute_group(b, g):
            lo = g * 16
            vals = x_loc[b, pl.ds(lo, 16)]
            tk = taken[pl.ds(lo, 16)]
            masked = jnp.where(tk == 0, vals, _NEG_INF)
            mx = jnp.max(masked)
            st1(g1val, (b, g), mx)
            st1(g1idx, (b, g), lo + ffs_scalar(masked == mx))
            s = lax.div(g, 16)
            sv = g1val[b, pl.ds(s * 16, 16)]
            st1(g2val, (b, s), jnp.max(sv))
            st1(dirty, (b,), 1)

        pltpu.sync_copy(x_hbm.at[pl.ds(base, apt)], x_loc)
        zf = jnp.zeros((16,), jnp.float32)
        zi = jnp.zeros((16,), jnp.int32)

        def zero_taken(i, _):
            taken[pl.ds(i * 16, 16)] = zi
            cnt[pl.ds(i * 16, 16)] = zi
            return 0

        lax.fori_loop(0, m // 16, zero_taken, 0)

        def zero_pi(i, _):
            b = lax.div(i, m // 16)
            o = lax.rem(i, m // 16)
            pi_loc[b, pl.ds(o * 16, 16)] = zf
            return 0

        lax.fori_loop(0, apt * (m // 16), zero_pi, 0)

        def build_blk(i, _):
            b = lax.div(i, nsuper)
            blk = lax.rem(i, nsuper)
            gbase = (blk * 16 + lane16) * 16
            bvec = jnp.full((16,), b, jnp.int32)
            run = jnp.full((16,), _NEG_INF, jnp.float32)
            argk = jnp.zeros((16,), jnp.int32)
            for k in range(16):
                v = plsc.load_gather(x_loc, [bvec, gbase + k])
                upd = v > run
                run = jnp.where(upd, v, run)
                argk = jnp.where(upd, k, argk)
            g1val[b, pl.ds(blk * 16, 16)] = run
            g1idx[b, pl.ds(blk * 16, 16)] = gbase + argk
            return 0

        lax.fori_loop(0, apt * nsuper, build_blk, 0)

        def build_g2(i, _):
            b = lax.div(i, nsuper)
            s = lax.rem(i, nsuper)
            sv = g1val[b, pl.ds(s * 16, 16)]
            st1(g2val, (b, s), jnp.max(sv))
            return 0

        lax.fori_loop(0, apt * nsuper, build_g2, 0)
        dirty[:] = jnp.ones((16,), jnp.int32)

        ones16 = jnp.ones((16,), jnp.int32)

        def main_loop(parity):
            def sr_cond(c):
                return c[0] < m

            def sr_body(c):
                done, par = c
                dv = dirty[:]
                for b in range(apt):
                    @pl.when(dv[b] != 0)
                    def _():
                        g2 = g2val[b, :]
                        s = ffs_scalar(g2 == jnp.max(g2))
                        grp = g1val[b, pl.ds(s * 16, 16)]
                        g = s * 16 + ffs_scalar(grp == jnp.max(grp))
                        st1(cand_loc, (b,), ld1(g1idx, (b, g)))

                dirty[:] = zi
                pltpu.sync_copy(cand_loc, cand_slab.at[par, wid])
                plsc.subcore_barrier()
                pltpu.sync_copy(cand_slab.at[par], rbuf)

                limit = jnp.minimum(done + n, m)

                def ch_cond(cc):
                    return cc[1] != 0

                def ch_body(cc):
                    u, _ = cc
                    p_vec = u + lane16
                    valid = p_vec < limit
                    a_vec = lax.bitwise_and(p_vec, n - 1)
                    items = plsc.load_gather(
                        rbuf,
                        [lax.shift_right_logical(a_vec, 2),
                         lax.bitwise_and(a_vec, 3)],
                        mask=valid,
                    )
                    tkn = plsc.load_gather(taken, [items], mask=valid)
                    plsc.addupdate_scatter(cnt, [items], ones16, mask=valid)
                    mult = plsc.load_gather(cnt, [items], mask=valid)
                    plsc.store_scatter(cnt, [items], zi, mask=valid)
                    bad = jnp.logical_and(
                        jnp.logical_and(
                            jnp.logical_or(tkn != 0, mult > 1), valid
                        ),
                        jnp.logical_or(lane16 > 0, u > done),
                    )
                    stop = jnp.min(jnp.where(bad, lane16, 16))
                    acc_mask = jnp.logical_and(lane16 < stop, valid)
                    plsc.store_scatter(taken, [items], ones16, mask=acc_mask)
                    for b in range(apt):
                        leaders = plsc.load_gather(
                            g1idx,
                            [jnp.full((16,), b, jnp.int32),
                             lax.shift_right_logical(items, 4)],
                            mask=acc_mask,
                        )
                        hit = jnp.logical_and(leaders == items, acc_mask)

                        def rep_cond(cur):
                            return jnp.max(cur) >= 0

                        def rep_body(cur):
                            it = jnp.max(cur)
                            recompute_group(b, lax.shift_right_logical(it, 4))
                            return jnp.where(cur == it, -1, cur)

                        lax.while_loop(
                            rep_cond, rep_body,
                            jnp.where(hit, items, -1),
                        )

                    nacc = jnp.minimum(stop, limit - u)
                    cont = jnp.logical_and(stop == 16, u + nacc < limit)
                    return u + nacc, jnp.where(cont, 1, 0).astype(jnp.int32)

                upto, _ = lax.while_loop(ch_cond, ch_body,
                                         (done, jnp.int32(1)))

                for b in range(apt):
                    p_b = done + lax.bitwise_and(base + b - done, n - 1)

                    @pl.when(p_b < upto)
                    def _():
                        st1(pi_loc, (b, ld1(cand_loc, (b,))), 1.0)

                return upto, 1 - par

            lax.while_loop(sr_cond, sr_body, (jnp.int32(0), parity))

        main_loop(jnp.int32(0))
        pltpu.sync_copy(pi_loc, out_hbm.at[pl.ds(base, apt)])

    run = pl.kernel(
        body,
        out_type=jax.ShapeDtypeStruct((n, m), jnp.float32),
        mesh=mesh,
        compiler_params=pltpu.CompilerParams(
            needs_layout_passes=False, use_tc_tiling_on_sc=False
        ),
        scratch_types=[
            pltpu.VMEM((apt, m), jnp.float32),
            pltpu.VMEM((apt, m), jnp.float32),
            pltpu.VMEM((m,), jnp.int32),
            pltpu.VMEM((m,), jnp.int32),
            pltpu.VMEM((apt, ngroups), jnp.float32),
            pltpu.VMEM((apt, ngroups), jnp.int32),
            pltpu.VMEM((apt, nsuper), jnp.float32),
            pltpu.VMEM((16,), jnp.int32),
            pltpu.VMEM((num_tiles, 16), jnp.int32),
            pltpu.VMEM((16,), jnp.int32),
            pltpu.VMEM_SHARED((2, num_tiles, 16), jnp.int32),
        ],
    )
    return run(x)


def kernel(X):
    if X.ndim == 2:
        return _round_robin_2d(X)
    return jnp.stack([_round_robin_2d(X[i]) for i in range(X.shape[0])])


# speedup vs baseline: 102.5054x; 1.0138x over previous
import functools

import jax
import jax.numpy as jnp
from jax import lax
from jax.experimental import pallas as pl
from jax.experimental.pallas import tpu as pltpu
from jax.experimental.pallas import tpu_sc as plsc

_NEG_INF = float("-inf")


def _round_robin_2d(x):
    n, m = x.shape
    num_tiles = 16
    apt = n // num_tiles
    ngroups = m // 16
    nsuper = ngroups // 16
    rounds = m // n

    mesh = plsc.VectorSubcoreMesh(
        core_axis_name="c", subcore_axis_name="s", num_cores=1
    )

    def body(x_hbm, out_hbm, x_loc, pi_loc, taken, cnt, g1val, g1idx, g2val,
             cand_loc, rbuf, cand_slab):
        wid = lax.axis_index("s")
        base = wid * apt
        lane16 = lax.iota(jnp.int32, 16)
        lane0 = lane16 == 0

        def st1(ref, idxs, val):
            plsc.store_scatter(
                ref,
                [jnp.full((16,), i, jnp.int32) for i in idxs],
                jnp.full((16,), val, ref.dtype),
                mask=lane0,
            )

        def ld1(ref, idxs):
            v = plsc.load_gather(
                ref,
                [jnp.full((16,), i, jnp.int32) for i in idxs],
                mask=lane0,
            )
            return v[0]

        def ffs_scalar(mask_vec):
            return jnp.max(plsc.all_reduce_ffs(mask_vec))

        def recompute_group(b, g):
            lo = g * 16
            vals = x_loc[b, pl.ds(lo, 16)]
            tk = taken[pl.ds(lo, 16)]
            masked = jnp.where(tk == 0, vals, _NEG_INF)
            mx = jnp.max(masked)
            st1(g1val, (b, g), mx)
            st1(g1idx, (b, g), lo + ffs_scalar(masked == mx))
            s = lax.div(g, 16)
            sv = g1val[b, pl.ds(s * 16, 16)]
            st1(g2val, (b, s), jnp.max(sv))

        pltpu.sync_copy(x_hbm.at[pl.ds(base, apt)], x_loc)
        zf = jnp.zeros((16,), jnp.float32)
        zi = jnp.zeros((16,), jnp.int32)

        def zero_taken(i, _):
            taken[pl.ds(i * 16, 16)] = zi
            cnt[pl.ds(i * 16, 16)] = zi
            return 0

        lax.fori_loop(0, m // 16, zero_taken, 0)

        def zero_pi(i, _):
            b = lax.div(i, m // 16)
            o = lax.rem(i, m // 16)
            pi_loc[b, pl.ds(o * 16, 16)] = zf
            return 0

        lax.fori_loop(0, apt * (m // 16), zero_pi, 0)

        def build_blk(i, _):
            b = lax.div(i, nsuper)
            blk = lax.rem(i, nsuper)
            gbase = (blk * 16 + lane16) * 16
            bvec = jnp.full((16,), b, jnp.int32)
            run = jnp.full((16,), _NEG_INF, jnp.float32)
            argk = jnp.zeros((16,), jnp.int32)
            for k in range(16):
                v = plsc.load_gather(x_loc, [bvec, gbase + k])
                upd = v > run
                run = jnp.where(upd, v, run)
                argk = jnp.where(upd, k, argk)
            g1val[b, pl.ds(blk * 16, 16)] = run
            g1idx[b, pl.ds(blk * 16, 16)] = gbase + argk
            return 0

        lax.fori_loop(0, apt * nsuper, build_blk, 0)

        def build_g2(i, _):
            b = lax.div(i, nsuper)
            s = lax.rem(i, nsuper)
            sv = g1val[b, pl.ds(s * 16, 16)]
            st1(g2val, (b, s), jnp.max(sv))
            return 0

        lax.fori_loop(0, apt * nsuper, build_g2, 0)

        ones16 = jnp.ones((16,), jnp.int32)

        def main_loop(parity):
            def sr_cond(c):
                return c[0] < m

            def sr_body(c):
                done, par = c
                for b in range(apt):
                    g2 = g2val[b, :]
                    s = ffs_scalar(g2 == jnp.max(g2))
                    grp = g1val[b, pl.ds(s * 16, 16)]
                    g = s * 16 + ffs_scalar(grp == jnp.max(grp))
                    st1(cand_loc, (b,), ld1(g1idx, (b, g)))

                pltpu.sync_copy(cand_loc, cand_slab.at[par, wid])
                plsc.subcore_barrier()
                pltpu.sync_copy(cand_slab.at[par], rbuf)

                limit = jnp.minimum(done + n, m)

                def ch_cond(cc):
                    return cc[1] != 0

                def ch_body(cc):
                    u, _ = cc
                    p_vec = u + lane16
                    valid = p_vec < limit
                    a_vec = lax.bitwise_and(p_vec, n - 1)
                    items = plsc.load_gather(
                        rbuf,
                        [lax.shift_right_logical(a_vec, 2),
                         lax.bitwise_and(a_vec, 3)],
                        mask=valid,
                    )
                    tkn = plsc.load_gather(taken, [items], mask=valid)
                    plsc.addupdate_scatter(cnt, [items], ones16, mask=valid)
                    mult = plsc.load_gather(cnt, [items], mask=valid)
                    plsc.store_scatter(cnt, [items], zi, mask=valid)
                    bad = jnp.logical_and(
                        jnp.logical_and(
                            jnp.logical_or(tkn != 0, mult > 1), valid
                        ),
                        jnp.logical_or(lane16 > 0, u > done),
                    )
                    stop = jnp.min(jnp.where(bad, lane16, 16))
                    acc_mask = jnp.logical_and(lane16 < stop, valid)
                    plsc.store_scatter(taken, [items], ones16, mask=acc_mask)
                    for b in range(apt):
                        leaders = plsc.load_gather(
                            g1idx,
                            [jnp.full((16,), b, jnp.int32),
                             lax.shift_right_logical(items, 4)],
                            mask=acc_mask,
                        )
                        hit = jnp.logical_and(leaders == items, acc_mask)

                        def rep_cond(cur):
                            return jnp.max(cur) >= 0

                        def rep_body(cur):
                            it = jnp.max(cur)
                            recompute_group(b, lax.shift_right_logical(it, 4))
                            return jnp.where(cur == it, -1, cur)

                        lax.while_loop(
                            rep_cond, rep_body,
                            jnp.where(hit, items, -1),
                        )

                    nacc = jnp.minimum(stop, limit - u)
                    cont = jnp.logical_and(stop == 16, u + nacc < limit)
                    return u + nacc, jnp.where(cont, 1, 0).astype(jnp.int32)

                upto, _ = lax.while_loop(ch_cond, ch_body,
                                         (done, jnp.int32(1)))

                p_vec = done + lax.bitwise_and(base + lane16 - done, n - 1)
                acc_b = jnp.logical_and(p_vec < upto, lane16 < apt)
                plsc.store_scatter(
                    pi_loc, [lane16, cand_loc[:]],
                    jnp.ones((16,), jnp.float32), mask=acc_b,
                )

                return upto, 1 - par

            lax.while_loop(sr_cond, sr_body, (jnp.int32(0), parity))

        main_loop(jnp.int32(0))
        pltpu.sync_copy(pi_loc, out_hbm.at[pl.ds(base, apt)])

    run = pl.kernel(
        body,
        out_type=jax.ShapeDtypeStruct((n, m), jnp.float32),
        mesh=mesh,
        compiler_params=pltpu.CompilerParams(
            needs_layout_passes=False, use_tc_tiling_on_sc=False
        ),
        scratch_types=[
            pltpu.VMEM((apt, m), jnp.float32),
            pltpu.VMEM((apt, m), jnp.float32),
            pltpu.VMEM((m,), jnp.int32),
            pltpu.VMEM((m,), jnp.int32),
            pltpu.VMEM((apt, ngroups), jnp.float32),
            pltpu.VMEM((apt, ngroups), jnp.int32),
            pltpu.VMEM((apt, nsuper), jnp.float32),
            pltpu.VMEM((16,), jnp.int32),
            pltpu.VMEM((num_tiles, 16), jnp.int32),
            pltpu.VMEM_SHARED((2, num_tiles, 16), jnp.int32),
        ],
    )
    return run(x)


def kernel(X):
    if X.ndim == 2:
        return _round_robin_2d(X)
    return jnp.stack([_round_robin_2d(X[i]) for i in range(X.shape[0])])


# speedup vs baseline: 131.6667x; 1.2845x over previous
import functools

import jax
import jax.numpy as jnp
from jax import lax
from jax.experimental import pallas as pl
from jax.experimental.pallas import tpu as pltpu
from jax.experimental.pallas import tpu_sc as plsc

_NEG_INF = float("-inf")


def _round_robin_2d(x):
    n, m = x.shape
    num_tiles = 16
    apt = n // num_tiles
    ngroups = m // 16
    nsuper = ngroups // 16
    rounds = m // n

    mesh = plsc.VectorSubcoreMesh(
        core_axis_name="c", subcore_axis_name="s", num_cores=1
    )

    def body(x_hbm, out_hbm, x_loc, pi_loc, taken, cnt, g1val, g1idx, g2val,
             cand_loc, rbuf, cand_slab):
        wid = lax.axis_index("s")
        base = wid * apt
        lane16 = lax.iota(jnp.int32, 16)
        lane0 = lane16 == 0

        def st1(ref, idxs, val):
            plsc.store_scatter(
                ref,
                [jnp.full((16,), i, jnp.int32) for i in idxs],
                jnp.full((16,), val, ref.dtype),
                mask=lane0,
            )

        def ld1(ref, idxs):
            v = plsc.load_gather(
                ref,
                [jnp.full((16,), i, jnp.int32) for i in idxs],
                mask=lane0,
            )
            return v[0]

        def ffs_scalar(mask_vec):
            return jnp.max(plsc.all_reduce_ffs(mask_vec))

        def recompute_group(b, g):
            lo = g * 16
            s = lax.div(g, 16)
            vals = x_loc[b, pl.ds(lo, 16)]
            tk = taken[pl.ds(lo, 16)]
            sv = g1val[b, pl.ds(s * 16, 16)]
            masked = jnp.where(tk == 0, vals, _NEG_INF)
            mx = jnp.max(masked)
            sv2 = jnp.where(lane16 == lax.rem(g, 16), mx, sv)
            st1(g1val, (b, g), mx)
            st1(g1idx, (b, g), lo + ffs_scalar(masked == mx))
            st1(g2val, (b, s), jnp.max(sv2))

        pltpu.sync_copy(x_hbm.at[pl.ds(base, apt)], x_loc)
        zf = jnp.zeros((16,), jnp.float32)
        zi = jnp.zeros((16,), jnp.int32)

        def zero_taken(i, _):
            taken[pl.ds(i * 16, 16)] = zi
            cnt[pl.ds(i * 16, 16)] = zi
            return 0

        lax.fori_loop(0, m // 16, zero_taken, 0)

        def zero_pi(i, _):
            b = lax.div(i, m // 16)
            o = lax.rem(i, m // 16)
            pi_loc[b, pl.ds(o * 16, 16)] = zf
            return 0

        lax.fori_loop(0, apt * (m // 16), zero_pi, 0)

        def build_blk(i, _):
            b = lax.div(i, nsuper)
            blk = lax.rem(i, nsuper)
            gbase = (blk * 16 + lane16) * 16
            bvec = jnp.full((16,), b, jnp.int32)
            run = jnp.full((16,), _NEG_INF, jnp.float32)
            argk = jnp.zeros((16,), jnp.int32)
            for k in range(16):
                v = plsc.load_gather(x_loc, [bvec, gbase + k])
                upd = v > run
                run = jnp.where(upd, v, run)
                argk = jnp.where(upd, k, argk)
            g1val[b, pl.ds(blk * 16, 16)] = run
            g1idx[b, pl.ds(blk * 16, 16)] = gbase + argk
            return 0

        lax.fori_loop(0, apt * nsuper, build_blk, 0)

        def build_g2(i, _):
            b = lax.div(i, nsuper)
            s = lax.rem(i, nsuper)
            sv = g1val[b, pl.ds(s * 16, 16)]
            st1(g2val, (b, s), jnp.max(sv))
            return 0

        lax.fori_loop(0, apt * nsuper, build_g2, 0)

        ones16 = jnp.ones((16,), jnp.int32)

        def main_loop(parity):
            def sr_cond(c):
                return c[0] < m

            def sr_body(c):
                done, par = c
                for b in range(apt):
                    g2 = g2val[b, :]
                    s = ffs_scalar(g2 == jnp.max(g2))
                    grp = g1val[b, pl.ds(s * 16, 16)]
                    g = s * 16 + ffs_scalar(grp == jnp.max(grp))
                    st1(cand_loc, (b,), ld1(g1idx, (b, g)))

                pltpu.sync_copy(cand_loc, cand_slab.at[par, wid])
                plsc.subcore_barrier()
                pltpu.sync_copy(cand_slab.at[par], rbuf)

                limit = jnp.minimum(done + n, m)

                def ch_cond(cc):
                    return cc[1] != 0

                def ch_body(cc):
                    u, _ = cc
                    p_vec = u + lane16
                    valid = p_vec < limit
                    a_vec = lax.bitwise_and(p_vec, n - 1)
                    items = plsc.load_gather(
                        rbuf,
                        [lax.shift_right_logical(a_vec, 2),
                         lax.bitwise_and(a_vec, 3)],
                        mask=valid,
                    )
                    tkn = plsc.load_gather(taken, [items], mask=valid)
                    plsc.addupdate_scatter(cnt, [items], ones16, mask=valid)
                    mult = plsc.load_gather(cnt, [items], mask=valid)
                    plsc.store_scatter(cnt, [items], zi, mask=valid)
                    bad = jnp.logical_and(
                        jnp.logical_and(
                            jnp.logical_or(tkn != 0, mult > 1), valid
                        ),
                        jnp.logical_or(lane16 > 0, u > done),
                    )
                    stop = jnp.min(jnp.where(bad, lane16, 16))
                    acc_mask = jnp.logical_and(lane16 < stop, valid)
                    plsc.store_scatter(taken, [items], ones16, mask=acc_mask)
                    for b in range(apt):
                        leaders = plsc.load_gather(
                            g1idx,
                            [jnp.full((16,), b, jnp.int32),
                             lax.shift_right_logical(items, 4)],
                            mask=acc_mask,
                        )
                        hit = jnp.logical_and(leaders == items, acc_mask)
                        cur0 = jnp.where(hit, items, -1)

                        def rep_cond(rc):
                            return rc[1] >= 0

                        def rep_body(rc):
                            cur, it = rc
                            recompute_group(b, lax.shift_right_logical(it, 4))
                            cur = jnp.where(cur == it, -1, cur)
                            return cur, jnp.max(cur)

                        lax.while_loop(rep_cond, rep_body,
                                       (cur0, jnp.max(cur0)))

                    nacc = jnp.minimum(stop, limit - u)
                    cont = jnp.logical_and(stop == 16, u + nacc < limit)
                    return u + nacc, jnp.where(cont, 1, 0).astype(jnp.int32)

                upto, _ = lax.while_loop(ch_cond, ch_body,
                                         (done, jnp.int32(1)))

                p_vec = done + lax.bitwise_and(base + lane16 - done, n - 1)
                acc_b = jnp.logical_and(p_vec < upto, lane16 < apt)
                plsc.store_scatter(
                    pi_loc, [lane16, cand_loc[:]],
                    jnp.ones((16,), jnp.float32), mask=acc_b,
                )

                return upto, 1 - par

            lax.while_loop(sr_cond, sr_body, (jnp.int32(0), parity))

        main_loop(jnp.int32(0))
        pltpu.sync_copy(pi_loc, out_hbm.at[pl.ds(base, apt)])

    run = pl.kernel(
        body,
        out_type=jax.ShapeDtypeStruct((n, m), jnp.float32),
        mesh=mesh,
        compiler_params=pltpu.CompilerParams(
            needs_layout_passes=False, use_tc_tiling_on_sc=False
        ),
        scratch_types=[
            pltpu.VMEM((apt, m), jnp.float32),
            pltpu.VMEM((apt, m), jnp.float32),
            pltpu.VMEM((m,), jnp.int32),
            pltpu.VMEM((m,), jnp.int32),
            pltpu.VMEM((apt, ngroups), jnp.float32),
            pltpu.VMEM((apt, ngroups), jnp.int32),
            pltpu.VMEM((apt, nsuper), jnp.float32),
            pltpu.VMEM((16,), jnp.int32),
            pltpu.VMEM((num_tiles, 16), jnp.int32),
            pltpu.VMEM_SHARED((2, num_tiles, 16), jnp.int32),
        ],
    )
    return run(x)


def kernel(X):
    if X.ndim == 2:
        return _round_robin_2d(X)
    return jnp.stack([_round_robin_2d(X[i]) for i in range(X.shape[0])])
